# baseline (speedup 1.0000x reference)
import functools

import jax
import jax.numpy as jnp
from jax.experimental import pallas as pl
from jax.experimental.pallas import tpu as pltpu

_EPS = 1e-5
_EDGE_BLK = 20000
_SUB = 2000


def _ln(x):
    m = jnp.mean(x, axis=-1, keepdims=True)
    c = x - m
    v = jnp.mean(c * c, axis=-1, keepdims=True)
    return c * jax.lax.rsqrt(v + _EPS)


def _proj(x, w, bias):
    return jnp.dot(x.astype(jnp.bfloat16), w,
                   preferred_element_type=jnp.float32) + bias


def _ln_proj_tiled(x_ref, out_ref, w, bias, sub):
    def one(j, carry):
        sl = pl.ds(j * sub, sub)
        out_ref[sl, :] = _proj(_ln(x_ref[sl, :]), w, bias)
        return carry
    jax.lax.fori_loop(0, x_ref.shape[0] // sub, one, 0, unroll=False)


def _fused_body(edge_ref, node_ref, q_ref,
                rel_w_ref, rel_b_ref, node_w_ref, node_b_ref,
                q_w_ref, q_b_ref,
                rel_out_ref, node_out_ref, q_out_ref, *, last_step):
    _ln_proj_tiled(edge_ref, rel_out_ref, rel_w_ref[:], rel_b_ref[:], _SUB)

    @pl.when(pl.program_id(0) == last_step)
    def _():
        _ln_proj_tiled(node_ref, node_out_ref, node_w_ref[:], node_b_ref[:], _SUB)
        q_out_ref[:] = _proj(q_ref[:], q_w_ref[:], q_b_ref[:])


def kernel(node_embeddings, edge_embeddings, question_emb,
           node_norm_g, node_norm_b, rel_norm_g, rel_norm_b,
           node_W, node_b, rel_W, rel_b, q_W, q_b):
    n_rows, d = node_embeddings.shape
    e_rows, _ = edge_embeddings.shape
    b_rows, _ = question_emb.shape
    h = node_W.shape[1]

    node_Wg = (node_norm_g[:, None] * node_W).astype(jnp.bfloat16)
    node_bias2 = (node_norm_b @ node_W + node_b).reshape(1, h)
    rel_Wg = (rel_norm_g[:, None] * rel_W).astype(jnp.bfloat16)
    rel_bias2 = (rel_norm_b @ rel_W + rel_b).reshape(1, h)

    grid = pl.cdiv(e_rows, _EDGE_BLK)
    const = lambda i: (0, 0)
    body = functools.partial(_fused_body, last_step=grid - 1)

    rel_out, node_out, q_out = pl.pallas_call(
        body,
        grid=(grid,),
        in_specs=[
            pl.BlockSpec((_EDGE_BLK, d), lambda i: (i, 0)),
            pl.BlockSpec((n_rows, d), const),
            pl.BlockSpec((b_rows, d), const),
            pl.BlockSpec((d, h), const),
            pl.BlockSpec((1, h), const),
            pl.BlockSpec((d, h), const),
            pl.BlockSpec((1, h), const),
            pl.BlockSpec((d, h), const),
            pl.BlockSpec((1, h), const),
        ],
        out_specs=[
            pl.BlockSpec((_EDGE_BLK, h), lambda i: (i, 0)),
            pl.BlockSpec((n_rows, h), const),
            pl.BlockSpec((b_rows, h), const),
        ],
        out_shape=[
            jax.ShapeDtypeStruct((e_rows, h), jnp.float32),
            jax.ShapeDtypeStruct((n_rows, h), jnp.float32),
            jax.ShapeDtypeStruct((b_rows, h), jnp.float32),
        ],
        compiler_params=pltpu.CompilerParams(
            dimension_semantics=("arbitrary",)),
    )(edge_embeddings, node_embeddings, question_emb,
      rel_Wg, rel_bias2, node_Wg, node_bias2,
      q_W.astype(jnp.bfloat16), q_b.reshape(1, h))

    return (node_out, rel_out, q_out)


# speedup vs baseline: 1.2182x; 1.2182x over previous
import functools

import jax
import jax.numpy as jnp
from jax.experimental import pallas as pl
from jax.experimental.pallas import tpu as pltpu

_EPS = 1e-5
_EDGE_BLK = 20000
_SUB = 5000


def _ln(x):
    m = jnp.mean(x, axis=-1, keepdims=True)
    c = x - m
    v = jnp.mean(c * c, axis=-1, keepdims=True)
    return c * jax.lax.rsqrt(v + _EPS)


def _proj(x, w, bias):
    return jnp.dot(x.astype(jnp.bfloat16), w,
                   preferred_element_type=jnp.float32) + bias


def _ln_proj_tiled(x_ref, out_ref, w, bias, sub):
    for j in range(x_ref.shape[0] // sub):
        sl = pl.ds(j * sub, sub)
        out_ref[sl, :] = _proj(_ln(x_ref[sl, :]), w, bias)


def _fused_body(edge_ref, node_ref, q_ref,
                rel_w_ref, rel_b_ref, node_w_ref, node_b_ref,
                q_w_ref, q_b_ref,
                rel_out_ref, node_out_ref, q_out_ref, *, last_step):
    _ln_proj_tiled(edge_ref, rel_out_ref, rel_w_ref[:], rel_b_ref[:], _SUB)

    @pl.when(pl.program_id(0) == last_step)
    def _():
        _ln_proj_tiled(node_ref, node_out_ref, node_w_ref[:], node_b_ref[:], _SUB)
        q_out_ref[:] = _proj(q_ref[:], q_w_ref[:], q_b_ref[:])


def kernel(node_embeddings, edge_embeddings, question_emb,
           node_norm_g, node_norm_b, rel_norm_g, rel_norm_b,
           node_W, node_b, rel_W, rel_b, q_W, q_b):
    n_rows, d = node_embeddings.shape
    e_rows, _ = edge_embeddings.shape
    b_rows, _ = question_emb.shape
    h = node_W.shape[1]

    node_Wg = (node_norm_g[:, None] * node_W).astype(jnp.bfloat16)
    node_bias2 = (node_norm_b @ node_W + node_b).reshape(1, h)
    rel_Wg = (rel_norm_g[:, None] * rel_W).astype(jnp.bfloat16)
    rel_bias2 = (rel_norm_b @ rel_W + rel_b).reshape(1, h)

    grid = pl.cdiv(e_rows, _EDGE_BLK)
    const = lambda i: (0, 0)
    body = functools.partial(_fused_body, last_step=grid - 1)

    rel_out, node_out, q_out = pl.pallas_call(
        body,
        grid=(grid,),
        in_specs=[
            pl.BlockSpec((_EDGE_BLK, d), lambda i: (i, 0)),
            pl.BlockSpec((n_rows, d), const),
            pl.BlockSpec((b_rows, d), const),
            pl.BlockSpec((d, h), const),
            pl.BlockSpec((1, h), const),
            pl.BlockSpec((d, h), const),
            pl.BlockSpec((1, h), const),
            pl.BlockSpec((d, h), const),
            pl.BlockSpec((1, h), const),
        ],
        out_specs=[
            pl.BlockSpec((_EDGE_BLK, h), lambda i: (i, 0)),
            pl.BlockSpec((n_rows, h), const),
            pl.BlockSpec((b_rows, h), const),
        ],
        out_shape=[
            jax.ShapeDtypeStruct((e_rows, h), jnp.float32),
            jax.ShapeDtypeStruct((n_rows, h), jnp.float32),
            jax.ShapeDtypeStruct((b_rows, h), jnp.float32),
        ],
        compiler_params=pltpu.CompilerParams(
            dimension_semantics=("arbitrary",)),
    )(edge_embeddings, node_embeddings, question_emb,
      rel_Wg, rel_bias2, node_Wg, node_bias2,
      q_W.astype(jnp.bfloat16), q_b.reshape(1, h))

    return (node_out, rel_out, q_out)


# speedup vs baseline: 1.3223x; 1.0854x over previous
import functools

import jax
import jax.numpy as jnp
from jax.experimental import pallas as pl
from jax.experimental.pallas import tpu as pltpu

_EPS = 1e-5


def _body(x_ref, w_ref, bias_ref, o_ref, *, use_ln):
    x = x_ref[:]
    if use_ln:
        m = jnp.mean(x, axis=-1, keepdims=True)
        c = x - m
        v = jnp.mean(c * c, axis=-1, keepdims=True)
        p = jnp.dot(c.astype(jnp.bfloat16), w_ref[:],
                    preferred_element_type=jnp.float32)
        o_ref[:] = p * jax.lax.rsqrt(v + _EPS) + bias_ref[:]
    else:
        o_ref[:] = jnp.dot(x.astype(jnp.bfloat16), w_ref[:],
                           preferred_element_type=jnp.float32) + bias_ref[:]


def _ln_proj(x, w_bf16, bias2, *, use_ln, blk):
    rows, d = x.shape
    h = w_bf16.shape[1]
    body = functools.partial(_body, use_ln=use_ln)
    return pl.pallas_call(
        body,
        grid=(pl.cdiv(rows, blk),),
        in_specs=[
            pl.BlockSpec((blk, d), lambda i: (i, 0)),
            pl.BlockSpec((d, h), lambda i: (0, 0)),
            pl.BlockSpec((1, h), lambda i: (0, 0)),
        ],
        out_specs=pl.BlockSpec((blk, h), lambda i: (i, 0)),
        out_shape=jax.ShapeDtypeStruct((rows, h), jnp.float32),
        compiler_params=pltpu.CompilerParams(
            dimension_semantics=("arbitrary",)),
    )(x, w_bf16, bias2.reshape(1, h))


def kernel(node_embeddings, edge_embeddings, question_emb,
           node_norm_g, node_norm_b, rel_norm_g, rel_norm_b,
           node_W, node_b, rel_W, rel_b, q_W, q_b):
    node_Wg = (node_norm_g[:, None] * node_W).astype(jnp.bfloat16)
    node_bias2 = node_norm_b @ node_W + node_b
    rel_Wg = (rel_norm_g[:, None] * rel_W).astype(jnp.bfloat16)
    rel_bias2 = rel_norm_b @ rel_W + rel_b

    node_tokens = _ln_proj(node_embeddings, node_Wg, node_bias2,
                           use_ln=True, blk=5000)
    relation_tokens = _ln_proj(edge_embeddings, rel_Wg, rel_bias2,
                               use_ln=True, blk=20000)
    question_tokens = _ln_proj(question_emb, q_W.astype(jnp.bfloat16), q_b,
                               use_ln=False, blk=1024)
    return (node_tokens, relation_tokens, question_tokens)
